# baseline (speedup 1.0000x reference)
import functools

import jax
import jax.numpy as jnp
from jax import lax
from jax.experimental import pallas as pl
from jax.experimental.pallas import tpu as pltpu
from jax.experimental.pallas import tpu_sc as plsc

DIM_ = 32
WPR = DIM_ // 4
NC = 2
NS = 16
NW = NC * NS
C = 128
G = 4


def _pack_body(x_ref, o_ref):
    xf = x_ref[...].astype(jnp.float32)
    r = lax.broadcasted_iota(jnp.int32, (DIM_, WPR), 0)
    c = lax.broadcasted_iota(jnp.int32, (DIM_, WPR), 1)
    b = []
    for k in range(4):
        sel = (r == c * 4 + k).astype(jnp.float32)
        b.append(jnp.dot(xf, sel,
                         preferred_element_type=jnp.float32).astype(jnp.int32))
    o_ref[...] = ((b[0] & 255) | ((b[1] & 255) << 8) | ((b[2] & 255) << 16)
                  | (b[3] << 24))


def _gather_body(n_steps, w_hbm, x_hbm, out_hbm, idx_v, raw_v, sems):
    wid = lax.axis_index("s") * NC + lax.axis_index("c")
    pltpu.sync_copy(x_hbm.at[pl.ds(wid * n_steps * G, n_steps * G)], idx_v)

    def step(i, carry):
        copies = []
        for g in range(G):
            copies.append(pltpu.make_async_copy(
                w_hbm.at[idx_v.at[i * G + g]],
                raw_v.at[pl.ds(g * C, C)], sems.at[g]))
        for cp in copies:
            cp.start()
        for cp in copies:
            cp.wait()
        pltpu.sync_copy(
            raw_v, out_hbm.at[pl.ds((wid * n_steps + i) * (G * C), G * C)])
        return carry

    lax.fori_loop(0, n_steps, step, 0)


def _dequant_body(L, x_ref, s_ref, o_ref):
    v = x_ref[...].astype(jnp.float32) * s_ref[0, 0]
    b_blk = x_ref.shape[0]
    o_ref[...] = v.reshape(b_blk, L, DIM_).transpose(1, 2, 0)


def kernel(x, weight, scale):
    B, L = x.shape
    vocab, dim = weight.shape
    n = B * L
    assert dim == DIM_ and n % (NW * C * G) == 0
    n_steps = n // (NW * C * G)

    rb = 8000
    assert vocab % rb == 0
    pack = pl.pallas_call(
        _pack_body,
        grid=(vocab // rb,),
        in_specs=[pl.BlockSpec((rb, dim), lambda i: (i, 0))],
        out_specs=pl.BlockSpec((rb, WPR), lambda i: (i, 0)),
        out_shape=jax.ShapeDtypeStruct((vocab, WPR), jnp.int32),
    )
    w32 = pack(weight)

    x2 = x.reshape(n // C, C)
    mesh = plsc.VectorSubcoreMesh(core_axis_name="c", subcore_axis_name="s",
                                  num_cores=NC, num_subcores=NS)
    gather = pl.kernel(
        functools.partial(_gather_body, n_steps),
        out_type=jax.ShapeDtypeStruct((n, WPR), jnp.int32),
        mesh=mesh,
        scratch_types=[
            pltpu.VMEM((n_steps * G, C), jnp.int32),
            pltpu.VMEM((G * C, WPR), jnp.int32),
            pltpu.SemaphoreType.DMA((G,)),
        ],
        compiler_params=pltpu.CompilerParams(needs_layout_passes=False,
                                             use_tc_tiling_on_sc=False),
    )
    g32 = gather(w32, x2)

    g8 = lax.bitcast_convert_type(g32, jnp.int8)
    b_blk = 1024
    g3 = g8.reshape(B, L * dim)
    dequant = pl.pallas_call(
        functools.partial(_dequant_body, L),
        grid=(B // b_blk,),
        in_specs=[
            pl.BlockSpec((b_blk, L * dim), lambda i: (i, 0)),
            pl.BlockSpec(memory_space=pltpu.SMEM),
        ],
        out_specs=pl.BlockSpec((L, dim, b_blk), lambda i: (0, 0, i)),
        out_shape=jax.ShapeDtypeStruct((L, dim, B), jnp.float32),
    )
    out = dequant(g3, scale.astype(jnp.float32).reshape(1, 1))
    return out.transpose(2, 0, 1)


# speedup vs baseline: 4.0953x; 4.0953x over previous
import functools

import jax
import jax.numpy as jnp
from jax import lax
from jax.experimental import pallas as pl
from jax.experimental.pallas import tpu as pltpu
from jax.experimental.pallas import tpu_sc as plsc

DIM_ = 32
NC = 2
NS = 16
NW = NC * NS
C = 128
G = 4


def _gather_body(n_steps, w_hbm, x_hbm, out_hbm, idx_v, raw_v, sems):
    wid = lax.axis_index("s") * NC + lax.axis_index("c")
    pltpu.sync_copy(x_hbm.at[pl.ds(wid * n_steps * G, n_steps * G)], idx_v)

    def step(i, carry):
        copies = []
        for g in range(G):
            copies.append(pltpu.make_async_copy(
                w_hbm.at[idx_v.at[i * G + g]],
                raw_v.at[pl.ds(g * C, C)], sems.at[g]))
        for cp in copies:
            cp.start()
        for cp in copies:
            cp.wait()
        pltpu.sync_copy(
            raw_v, out_hbm.at[pl.ds((wid * n_steps + i) * (G * C), G * C)])
        return carry

    lax.fori_loop(0, n_steps, step, 0)


def _dequant_body(L, x_ref, s_ref, o_ref):
    v = x_ref[...].astype(jnp.float32) * s_ref[0, 0]
    b_blk = x_ref.shape[0]
    o_ref[...] = v.reshape(b_blk, L, DIM_).transpose(1, 2, 0)


def kernel(x, weight, scale):
    B, L = x.shape
    vocab, dim = weight.shape
    n = B * L
    assert dim == DIM_ and n % (NW * C * G) == 0
    n_steps = n // (NW * C * G)

    x2 = x.reshape(n // C, C)

    mesh = plsc.VectorSubcoreMesh(core_axis_name="c", subcore_axis_name="s",
                                  num_cores=NC, num_subcores=NS)
    gather = pl.kernel(
        functools.partial(_gather_body, n_steps),
        out_type=jax.ShapeDtypeStruct((n, dim), jnp.int8),
        mesh=mesh,
        scratch_types=[
            pltpu.VMEM((n_steps * G, C), jnp.int32),
            pltpu.VMEM((G * C, dim), jnp.int8),
            pltpu.SemaphoreType.DMA((G,)),
        ],
        compiler_params=pltpu.CompilerParams(needs_layout_passes=False,
                                             use_tc_tiling_on_sc=False),
    )
    g8 = gather(weight, x2)

    b_blk = 1024
    g3 = g8.reshape(B, L * dim)
    dequant = pl.pallas_call(
        functools.partial(_dequant_body, L),
        grid=(B // b_blk,),
        in_specs=[
            pl.BlockSpec((b_blk, L * dim), lambda i: (i, 0)),
            pl.BlockSpec(memory_space=pltpu.SMEM),
        ],
        out_specs=pl.BlockSpec((L, dim, b_blk), lambda i: (0, 0, i)),
        out_shape=jax.ShapeDtypeStruct((L, dim, B), jnp.float32),
    )
    out = dequant(g3, scale.astype(jnp.float32).reshape(1, 1))
    return out.transpose(2, 0, 1)
